# baseline (speedup 1.0000x reference)
import functools

import jax
import jax.numpy as jnp
from jax import lax
from jax.experimental import pallas as pl
from jax.experimental.pallas import tpu as pltpu
from jax.experimental.pallas import tpu_sc as plsc

N = 10000
E = 320000
NPAD = 10240

NC = 2
NS = 16
NW = NC * NS
CHUNK = 128
NCHUNK = 80
EP = NCHUNK * CHUNK
EPAD = NW * EP
ZROWS = 16
ROWS_PER_TILE = NPAD // NS

_MESH = plsc.VectorSubcoreMesh(core_axis_name="c", subcore_axis_name="s")


@functools.partial(
    pl.kernel,
    out_type=jax.ShapeDtypeStruct((NC, NPAD), jnp.float32),
    mesh=_MESH,
    scratch_types=[
        pltpu.VMEM((NCHUNK, CHUNK), jnp.int32),
        pltpu.VMEM((CHUNK,), jnp.float32),
        pltpu.VMEM((ROWS_PER_TILE,), jnp.float32),
        pltpu.VMEM_SHARED((NPAD,), jnp.float32),
        pltpu.SemaphoreType.DMA,
    ],
)
def _deg_kernel(dst_hbm, out_hbm, dst_idx, ones_v, zbuf, acc, isem):
    c = lax.axis_index("c")
    sub = lax.axis_index("s")
    wid = sub * NC + c

    icp = pltpu.async_copy(dst_hbm.at[pl.ds(wid * NCHUNK, NCHUNK)], dst_idx,
                           isem)

    def fill(i, _):
        ones_v[pl.ds(i * 16, 16)] = jnp.ones((16,), jnp.float32)
        return 0

    lax.fori_loop(0, CHUNK // 16, fill, 0)

    def zfill(i, _):
        zbuf[pl.ds(i * 16, 16)] = jnp.zeros((16,), jnp.float32)
        return 0

    lax.fori_loop(0, ROWS_PER_TILE // 16, zfill, 0)
    pltpu.sync_copy(zbuf, acc.at[pl.ds(sub * ROWS_PER_TILE, ROWS_PER_TILE)])
    icp.wait()
    plsc.subcore_barrier()

    def body(i, _):
        pltpu.sync_copy(ones_v, acc.at[dst_idx.at[i]], add=True)
        return 0

    lax.fori_loop(0, NCHUNK, body, 0)
    plsc.subcore_barrier()
    pltpu.sync_copy(
        acc.at[pl.ds(sub * ROWS_PER_TILE, ROWS_PER_TILE)],
        out_hbm.at[c, pl.ds(sub * ROWS_PER_TILE, ROWS_PER_TILE)],
    )


def _make_sc_apply(D):
    @functools.partial(
        pl.kernel,
        out_type=jax.ShapeDtypeStruct((NC, NPAD, D), jnp.float32),
        mesh=_MESH,
        scratch_types=[
            pltpu.VMEM((NCHUNK, CHUNK), jnp.int32),
            pltpu.VMEM((2, 1, CHUNK), jnp.int32),
            pltpu.VMEM((2, CHUNK, D), jnp.float32),
            pltpu.VMEM((ZROWS, D), jnp.float32),
            pltpu.VMEM_SHARED((NPAD, D), jnp.float32),
            pltpu.SemaphoreType.DMA,
            pltpu.SemaphoreType.DMA,
            pltpu.SemaphoreType.DMA,
            pltpu.SemaphoreType.DMA,
            pltpu.SemaphoreType.DMA,
            pltpu.SemaphoreType.DMA,
        ],
        compiler_params=pltpu.CompilerParams(use_tc_tiling_on_sc=False),
    )
    def sc_apply(p_hbm, src_hbm, dst3_hbm, out_hbm, src_idx, dst_idx, rows,
                 zbuf, acc, g0, g1, s0, s1, d0, d1):
        c = lax.axis_index("c")
        sub = lax.axis_index("s")
        wid = sub * NC + c
        gsem = (g0, g1)
        ssem = (s0, s1)
        dsem = (d0, d1)

        def gstart(ci, b):
            pltpu.async_copy(p_hbm.at[src_idx.at[ci]], rows.at[b], gsem[b])

        def gwait(b):
            pltpu.make_async_copy(
                p_hbm.at[src_idx.at[0]], rows.at[b], gsem[b]
            ).wait()

        def dstart(ci, b):
            pltpu.async_copy(dst3_hbm.at[wid * NCHUNK + ci], dst_idx.at[b],
                             dsem[b])

        def dwait(b):
            pltpu.make_async_copy(
                dst3_hbm.at[0], dst_idx.at[b], dsem[b]
            ).wait()

        def sstart(b):
            pltpu.async_copy(rows.at[b], acc.at[dst_idx.at[b, 0]], ssem[b],
                             add=True)

        def swait(b):
            pltpu.make_async_copy(
                rows.at[b], acc.at[dst_idx.at[b, 0]], ssem[b]
            ).wait()

        icp0 = pltpu.async_copy(
            src_hbm.at[pl.ds(wid * NCHUNK, NCHUNK)], src_idx, g0)

        nz = (ZROWS * D) // 16

        def zfill(i, _):
            r = i // (D // 16)
            col = (i % (D // 16)) * 16
            zbuf[r, pl.ds(col, 16)] = jnp.zeros((16,), jnp.float32)
            return 0

        lax.fori_loop(0, nz, zfill, 0)

        dstart(0, 0)
        dstart(1, 1)
        icp0.wait()
        gstart(0, 0)
        gstart(1, 1)

        def zcopy(i, _):
            pltpu.sync_copy(
                zbuf, acc.at[pl.ds(sub * ROWS_PER_TILE + i * ZROWS, ZROWS)]
            )
            return 0

        lax.fori_loop(0, ROWS_PER_TILE // ZROWS, zcopy, 0)
        plsc.subcore_barrier()

        def stage(i, b, more):
            gwait(b)
            dwait(b)
            sstart(b)
            swait(b)
            if more:
                gstart(i + 2, b)
                dstart(i + 2, b)

        def step(k, _):
            i = 2 * k
            stage(i, 0, True)
            stage(i + 1, 1, True)
            return 0

        lax.fori_loop(0, NCHUNK // 2 - 2, step, 0)
        t = NCHUNK - 4
        stage(t, 0, True)
        stage(t + 1, 1, True)
        stage(t + 2, 0, False)
        stage(t + 3, 1, False)
        plsc.subcore_barrier()
        pltpu.sync_copy(
            acc.at[pl.ds(sub * ROWS_PER_TILE, ROWS_PER_TILE)],
            out_hbm.at[c, pl.ds(sub * ROWS_PER_TILE, ROWS_PER_TILE)],
        )

    return sc_apply


_sc_apply = {D: _make_sc_apply(D) for D in (128, 64, 32)}


_RB = 1280
_GRID = NPAD // _RB


def _tc0_body(x_ref, w_ref, d0_ref, d1_ref, p_ref, dis_ref):
    dis = lax.rsqrt(d0_ref[...] + d1_ref[...] + 1.0)
    h = jnp.dot(x_ref[...], w_ref[...], preferred_element_type=jnp.float32)
    p_ref[...] = dis * h
    dis_ref[...] = dis


def _tc0(xpad, W1, deg0, deg1):
    D = W1.shape[1]
    return pl.pallas_call(
        _tc0_body,
        grid=(_GRID,),
        in_specs=[
            pl.BlockSpec((_RB, xpad.shape[1]), lambda i: (i, 0)),
            pl.BlockSpec(W1.shape, lambda i: (0, 0)),
            pl.BlockSpec((_RB, 1), lambda i: (i, 0)),
            pl.BlockSpec((_RB, 1), lambda i: (i, 0)),
        ],
        out_specs=[
            pl.BlockSpec((_RB, D), lambda i: (i, 0)),
            pl.BlockSpec((_RB, 1), lambda i: (i, 0)),
        ],
        out_shape=[
            jax.ShapeDtypeStruct((NPAD, D), jnp.float32),
            jax.ShapeDtypeStruct((NPAD, 1), jnp.float32),
        ],
    )(xpad, W1, deg0, deg1)


def _tc_layer_body(s0_ref, s1_ref, p_ref, dis_ref, b_ref, w_ref, out_ref):
    dis = dis_ref[...]
    a = dis * (s0_ref[...] + s1_ref[...] + p_ref[...]) + b_ref[...]
    h = jnp.maximum(a, 0.0)
    out_ref[...] = dis * jnp.dot(
        h, w_ref[...], preferred_element_type=jnp.float32
    )


def _tc_layer(s0, s1, p, dis, b, Wn):
    Din, Dout = Wn.shape
    return pl.pallas_call(
        _tc_layer_body,
        grid=(_GRID,),
        in_specs=[
            pl.BlockSpec((_RB, Din), lambda i: (i, 0)),
            pl.BlockSpec((_RB, Din), lambda i: (i, 0)),
            pl.BlockSpec((_RB, Din), lambda i: (i, 0)),
            pl.BlockSpec((_RB, 1), lambda i: (i, 0)),
            pl.BlockSpec((1, Din), lambda i: (0, 0)),
            pl.BlockSpec((Din, Dout), lambda i: (0, 0)),
        ],
        out_specs=pl.BlockSpec((_RB, Dout), lambda i: (i, 0)),
        out_shape=jax.ShapeDtypeStruct((NPAD, Dout), jnp.float32),
    )(s0, s1, p, dis, b.reshape(1, Din), Wn)


def _tc_final_body(s0_ref, s1_ref, p_ref, dis_ref, b_ref, out_ref):
    out_ref[...] = (
        dis_ref[...] * (s0_ref[...] + s1_ref[...] + p_ref[...]) + b_ref[...]
    )


def _tc_final(s0, s1, p, dis, b):
    D = p.shape[1]
    return pl.pallas_call(
        _tc_final_body,
        grid=(_GRID,),
        in_specs=[
            pl.BlockSpec((_RB, D), lambda i: (i, 0)),
            pl.BlockSpec((_RB, D), lambda i: (i, 0)),
            pl.BlockSpec((_RB, D), lambda i: (i, 0)),
            pl.BlockSpec((_RB, 1), lambda i: (i, 0)),
            pl.BlockSpec((1, D), lambda i: (0, 0)),
        ],
        out_specs=pl.BlockSpec((_RB, D), lambda i: (i, 0)),
        out_shape=jax.ShapeDtypeStruct((NPAD, D), jnp.float32),
    )(s0, s1, p, dis, b.reshape(1, D))


def kernel(x, edge_index, W1, b1, W2, b2, W3, b3, W4, b4):
    extra = EPAD - E
    src = jnp.concatenate(
        [edge_index[0], jnp.zeros((extra,), jnp.int32)]
    ).reshape(EPAD // CHUNK, CHUNK)
    dst_flat = jnp.concatenate(
        [edge_index[1], jnp.full((extra,), NPAD - 1, jnp.int32)]
    )
    dst = dst_flat.reshape(EPAD // CHUNK, CHUNK)
    dst3 = dst_flat.reshape(EPAD // CHUNK, 1, CHUNK)
    xpad = jnp.pad(x, ((0, NPAD - N), (0, 0)))

    degp = _deg_kernel(dst)
    deg0 = degp[0].reshape(NPAD, 1)
    deg1 = degp[1].reshape(NPAD, 1)

    p1, dis = _tc0(xpad, W1, deg0, deg1)
    s1 = _sc_apply[128](p1, src, dst3)
    p2 = _tc_layer(s1[0], s1[1], p1, dis, b1, W2)
    s2 = _sc_apply[128](p2, src, dst3)
    p3 = _tc_layer(s2[0], s2[1], p2, dis, b2, W3)
    s3 = _sc_apply[64](p3, src, dst3)
    p4 = _tc_layer(s3[0], s3[1], p3, dis, b3, W4)
    s4 = _sc_apply[32](p4, src, dst3)
    z = _tc_final(s4[0], s4[1], p4, dis, b4)
    return z[:N]


# speedup vs baseline: 1.1728x; 1.1728x over previous
import functools

import jax
import jax.numpy as jnp
from jax import lax
from jax.experimental import pallas as pl
from jax.experimental.pallas import tpu as pltpu
from jax.experimental.pallas import tpu_sc as plsc

N = 10000
E = 320000
NPAD = 10240

NC = 2
NS = 16
NW = NC * NS
CHUNK = 128
NCHUNK = 80
EP = NCHUNK * CHUNK
EPAD = NW * EP
ZROWS = 16
ROWS_PER_TILE = NPAD // NS

_MESH = plsc.VectorSubcoreMesh(core_axis_name="c", subcore_axis_name="s")


@functools.partial(
    pl.kernel,
    out_type=jax.ShapeDtypeStruct((NC, NPAD), jnp.float32),
    mesh=_MESH,
    scratch_types=[
        pltpu.VMEM((NCHUNK, CHUNK), jnp.int32),
        pltpu.VMEM((CHUNK,), jnp.float32),
        pltpu.VMEM((ROWS_PER_TILE,), jnp.float32),
        pltpu.VMEM_SHARED((NPAD,), jnp.float32),
        pltpu.SemaphoreType.DMA,
    ],
)
def _deg_kernel(dst_hbm, out_hbm, dst_idx, ones_v, zbuf, acc, isem):
    c = lax.axis_index("c")
    sub = lax.axis_index("s")
    wid = sub * NC + c

    icp = pltpu.async_copy(dst_hbm.at[pl.ds(wid * NCHUNK, NCHUNK)], dst_idx,
                           isem)

    def fill(i, _):
        ones_v[pl.ds(i * 16, 16)] = jnp.ones((16,), jnp.float32)
        return 0

    lax.fori_loop(0, CHUNK // 16, fill, 0)

    def zfill(i, _):
        zbuf[pl.ds(i * 16, 16)] = jnp.zeros((16,), jnp.float32)
        return 0

    lax.fori_loop(0, ROWS_PER_TILE // 16, zfill, 0)
    pltpu.sync_copy(zbuf, acc.at[pl.ds(sub * ROWS_PER_TILE, ROWS_PER_TILE)])
    icp.wait()
    plsc.subcore_barrier()

    def body(i, _):
        pltpu.async_copy(ones_v, acc.at[dst_idx.at[i]], isem, add=True)
        return 0

    lax.fori_loop(0, NCHUNK, body, 0)

    def drain(i, _):
        pltpu.make_async_copy(ones_v, acc.at[dst_idx.at[0]], isem).wait()
        return 0

    lax.fori_loop(0, NCHUNK, drain, 0)
    plsc.subcore_barrier()
    pltpu.sync_copy(
        acc.at[pl.ds(sub * ROWS_PER_TILE, ROWS_PER_TILE)],
        out_hbm.at[c, pl.ds(sub * ROWS_PER_TILE, ROWS_PER_TILE)],
    )


def _make_sc_apply(D):
    @functools.partial(
        pl.kernel,
        out_type=jax.ShapeDtypeStruct((NC, NPAD, D), jnp.float32),
        mesh=_MESH,
        scratch_types=[
            pltpu.VMEM((NCHUNK, CHUNK), jnp.int32),
            pltpu.VMEM((2, 1, CHUNK), jnp.int32),
            pltpu.VMEM((2, CHUNK, D), jnp.float32),
            pltpu.VMEM((ZROWS, D), jnp.float32),
            pltpu.VMEM_SHARED((NPAD, D), jnp.float32),
            pltpu.SemaphoreType.DMA,
            pltpu.SemaphoreType.DMA,
            pltpu.SemaphoreType.DMA,
            pltpu.SemaphoreType.DMA,
            pltpu.SemaphoreType.DMA,
            pltpu.SemaphoreType.DMA,
        ],
        compiler_params=pltpu.CompilerParams(use_tc_tiling_on_sc=False),
    )
    def sc_apply(p_hbm, src_hbm, dst3_hbm, out_hbm, src_idx, dst_idx, rows,
                 zbuf, acc, g0, g1, s0, s1, d0, d1):
        c = lax.axis_index("c")
        sub = lax.axis_index("s")
        wid = sub * NC + c
        gsem = (g0, g1)
        ssem = (s0, s1)
        dsem = (d0, d1)

        def gstart(ci, b):
            pltpu.async_copy(p_hbm.at[src_idx.at[ci]], rows.at[b], gsem[b])

        def gwait(b):
            pltpu.make_async_copy(
                p_hbm.at[src_idx.at[0]], rows.at[b], gsem[b]
            ).wait()

        def dstart(ci, b):
            pltpu.async_copy(dst3_hbm.at[wid * NCHUNK + ci], dst_idx.at[b],
                             dsem[b])

        def dwait(b):
            pltpu.make_async_copy(
                dst3_hbm.at[0], dst_idx.at[b], dsem[b]
            ).wait()

        def sstart(b):
            pltpu.async_copy(rows.at[b], acc.at[dst_idx.at[b, 0]], ssem[b],
                             add=True)

        def swait(b):
            pltpu.make_async_copy(
                rows.at[b], acc.at[dst_idx.at[b, 0]], ssem[b]
            ).wait()

        icp0 = pltpu.async_copy(
            src_hbm.at[pl.ds(wid * NCHUNK, NCHUNK)], src_idx, g0)

        nz = (ZROWS * D) // 16

        def zfill(i, _):
            r = i // (D // 16)
            col = (i % (D // 16)) * 16
            zbuf[r, pl.ds(col, 16)] = jnp.zeros((16,), jnp.float32)
            return 0

        lax.fori_loop(0, nz, zfill, 0)

        dstart(0, 0)
        dstart(1, 1)
        icp0.wait()
        gstart(0, 0)
        gstart(1, 1)

        def zcopy(i, _):
            pltpu.sync_copy(
                zbuf, acc.at[pl.ds(sub * ROWS_PER_TILE + i * ZROWS, ZROWS)]
            )
            return 0

        lax.fori_loop(0, ROWS_PER_TILE // ZROWS, zcopy, 0)
        plsc.subcore_barrier()

        def stage(i, b, more):
            gwait(b)
            dwait(b)
            sstart(b)
            swait(b)
            if more:
                gstart(i + 2, b)
                dstart(i + 2, b)

        def step(k, _):
            i = 2 * k
            stage(i, 0, True)
            stage(i + 1, 1, True)
            return 0

        lax.fori_loop(0, NCHUNK // 2 - 2, step, 0)
        t = NCHUNK - 4
        stage(t, 0, True)
        stage(t + 1, 1, True)
        stage(t + 2, 0, False)
        stage(t + 3, 1, False)
        plsc.subcore_barrier()
        pltpu.sync_copy(
            acc.at[pl.ds(sub * ROWS_PER_TILE, ROWS_PER_TILE)],
            out_hbm.at[c, pl.ds(sub * ROWS_PER_TILE, ROWS_PER_TILE)],
        )

    return sc_apply


_sc_apply = {D: _make_sc_apply(D) for D in (128, 64, 32)}


_RB = 1280
_GRID = NPAD // _RB


def _tc0_body(x_ref, w_ref, d0_ref, d1_ref, p_ref, dis_ref):
    dis = lax.rsqrt(d0_ref[...] + d1_ref[...] + 1.0)
    h = jnp.dot(x_ref[...], w_ref[...], preferred_element_type=jnp.float32)
    p_ref[...] = dis * h
    dis_ref[...] = dis


def _tc0(xpad, W1, deg0, deg1):
    D = W1.shape[1]
    return pl.pallas_call(
        _tc0_body,
        grid=(_GRID,),
        in_specs=[
            pl.BlockSpec((_RB, xpad.shape[1]), lambda i: (i, 0)),
            pl.BlockSpec(W1.shape, lambda i: (0, 0)),
            pl.BlockSpec((_RB, 1), lambda i: (i, 0)),
            pl.BlockSpec((_RB, 1), lambda i: (i, 0)),
        ],
        out_specs=[
            pl.BlockSpec((_RB, D), lambda i: (i, 0)),
            pl.BlockSpec((_RB, 1), lambda i: (i, 0)),
        ],
        out_shape=[
            jax.ShapeDtypeStruct((NPAD, D), jnp.float32),
            jax.ShapeDtypeStruct((NPAD, 1), jnp.float32),
        ],
    )(xpad, W1, deg0, deg1)


def _tc_layer_body(s0_ref, s1_ref, p_ref, dis_ref, b_ref, w_ref, out_ref):
    dis = dis_ref[...]
    a = dis * (s0_ref[...] + s1_ref[...] + p_ref[...]) + b_ref[...]
    h = jnp.maximum(a, 0.0)
    out_ref[...] = dis * jnp.dot(
        h, w_ref[...], preferred_element_type=jnp.float32
    )


def _tc_layer(s0, s1, p, dis, b, Wn):
    Din, Dout = Wn.shape
    return pl.pallas_call(
        _tc_layer_body,
        grid=(_GRID,),
        in_specs=[
            pl.BlockSpec((_RB, Din), lambda i: (i, 0)),
            pl.BlockSpec((_RB, Din), lambda i: (i, 0)),
            pl.BlockSpec((_RB, Din), lambda i: (i, 0)),
            pl.BlockSpec((_RB, 1), lambda i: (i, 0)),
            pl.BlockSpec((1, Din), lambda i: (0, 0)),
            pl.BlockSpec((Din, Dout), lambda i: (0, 0)),
        ],
        out_specs=pl.BlockSpec((_RB, Dout), lambda i: (i, 0)),
        out_shape=jax.ShapeDtypeStruct((NPAD, Dout), jnp.float32),
    )(s0, s1, p, dis, b.reshape(1, Din), Wn)


def _tc_final_body(s0_ref, s1_ref, p_ref, dis_ref, b_ref, out_ref):
    out_ref[...] = (
        dis_ref[...] * (s0_ref[...] + s1_ref[...] + p_ref[...]) + b_ref[...]
    )


def _tc_final(s0, s1, p, dis, b):
    D = p.shape[1]
    return pl.pallas_call(
        _tc_final_body,
        grid=(_GRID,),
        in_specs=[
            pl.BlockSpec((_RB, D), lambda i: (i, 0)),
            pl.BlockSpec((_RB, D), lambda i: (i, 0)),
            pl.BlockSpec((_RB, D), lambda i: (i, 0)),
            pl.BlockSpec((_RB, 1), lambda i: (i, 0)),
            pl.BlockSpec((1, D), lambda i: (0, 0)),
        ],
        out_specs=pl.BlockSpec((_RB, D), lambda i: (i, 0)),
        out_shape=jax.ShapeDtypeStruct((NPAD, D), jnp.float32),
    )(s0, s1, p, dis, b.reshape(1, D))


def kernel(x, edge_index, W1, b1, W2, b2, W3, b3, W4, b4):
    extra = EPAD - E
    src = jnp.concatenate(
        [edge_index[0], jnp.zeros((extra,), jnp.int32)]
    ).reshape(EPAD // CHUNK, CHUNK)
    trash = N + (jnp.arange(extra, dtype=jnp.int32) % (NPAD - N))
    dst_flat = jnp.concatenate([edge_index[1], trash])
    dst = dst_flat.reshape(EPAD // CHUNK, CHUNK)
    dst3 = dst_flat.reshape(EPAD // CHUNK, 1, CHUNK)
    xpad = jnp.pad(x, ((0, NPAD - N), (0, 0)))

    degp = _deg_kernel(dst)
    deg0 = degp[0].reshape(NPAD, 1)
    deg1 = degp[1].reshape(NPAD, 1)

    p1, dis = _tc0(xpad, W1, deg0, deg1)
    s1 = _sc_apply[128](p1, src, dst3)
    p2 = _tc_layer(s1[0], s1[1], p1, dis, b1, W2)
    s2 = _sc_apply[128](p2, src, dst3)
    p3 = _tc_layer(s2[0], s2[1], p2, dis, b2, W3)
    s3 = _sc_apply[64](p3, src, dst3)
    p4 = _tc_layer(s3[0], s3[1], p3, dis, b3, W4)
    s4 = _sc_apply[32](p4, src, dst3)
    z = _tc_final(s4[0], s4[1], p4, dis, b4)
    return z[:N]


# speedup vs baseline: 1.2767x; 1.0886x over previous
import functools

import jax
import jax.numpy as jnp
from jax import lax
from jax.experimental import pallas as pl
from jax.experimental.pallas import tpu as pltpu
from jax.experimental.pallas import tpu_sc as plsc

N = 10000
E = 320000
NPAD = 10240

NC = 2
NS = 16
NW = NC * NS
CHUNK = 128
NCHUNK = 80
EP = NCHUNK * CHUNK
EPAD = NW * EP
ZROWS = 16
ROWS_PER_TILE = NPAD // NS
K0 = 120
K1 = 40

_MESH = plsc.VectorSubcoreMesh(core_axis_name="c", subcore_axis_name="s")


@functools.partial(
    pl.kernel,
    out_type=jax.ShapeDtypeStruct((NC, NPAD), jnp.float32),
    mesh=_MESH,
    scratch_types=[
        pltpu.VMEM((NCHUNK, CHUNK), jnp.int32),
        pltpu.VMEM((CHUNK,), jnp.float32),
        pltpu.VMEM((ROWS_PER_TILE,), jnp.float32),
        pltpu.VMEM_SHARED((NPAD,), jnp.float32),
        pltpu.SemaphoreType.DMA,
    ],
)
def _deg_kernel(dst_hbm, out_hbm, dst_idx, ones_v, zbuf, acc, isem):
    c = lax.axis_index("c")
    sub = lax.axis_index("s")
    wid = sub * NC + c

    icp = pltpu.async_copy(dst_hbm.at[pl.ds(wid * NCHUNK, NCHUNK)], dst_idx,
                           isem)

    def fill(i, _):
        ones_v[pl.ds(i * 16, 16)] = jnp.ones((16,), jnp.float32)
        return 0

    lax.fori_loop(0, CHUNK // 16, fill, 0)

    def zfill(i, _):
        zbuf[pl.ds(i * 16, 16)] = jnp.zeros((16,), jnp.float32)
        return 0

    lax.fori_loop(0, ROWS_PER_TILE // 16, zfill, 0)
    pltpu.sync_copy(zbuf, acc.at[pl.ds(sub * ROWS_PER_TILE, ROWS_PER_TILE)])
    icp.wait()
    plsc.subcore_barrier()

    def body(i, _):
        pltpu.async_copy(ones_v, acc.at[dst_idx.at[i]], isem, add=True)
        return 0

    lax.fori_loop(0, NCHUNK, body, 0)

    def drain(i, _):
        pltpu.make_async_copy(ones_v, acc.at[dst_idx.at[0]], isem).wait()
        return 0

    lax.fori_loop(0, NCHUNK, drain, 0)
    plsc.subcore_barrier()
    pltpu.sync_copy(
        acc.at[pl.ds(sub * ROWS_PER_TILE, ROWS_PER_TILE)],
        out_hbm.at[c, pl.ds(sub * ROWS_PER_TILE, ROWS_PER_TILE)],
    )


def _make_sc_apply(D):
    @functools.partial(
        pl.kernel,
        out_type=jax.ShapeDtypeStruct((NC, NPAD, D), jnp.float32),
        mesh=_MESH,
        scratch_types=[
            pltpu.VMEM((2, 1, CHUNK), jnp.int32),
            pltpu.VMEM((2, 1, CHUNK), jnp.int32),
            pltpu.VMEM((2, CHUNK, D), jnp.float32),
            pltpu.VMEM((ZROWS, D), jnp.float32),
            pltpu.VMEM_SHARED((NPAD, D), jnp.float32),
            pltpu.SemaphoreType.DMA,
            pltpu.SemaphoreType.DMA,
            pltpu.SemaphoreType.DMA,
            pltpu.SemaphoreType.DMA,
            pltpu.SemaphoreType.DMA,
            pltpu.SemaphoreType.DMA,
            pltpu.SemaphoreType.DMA,
            pltpu.SemaphoreType.DMA,
        ],
        compiler_params=pltpu.CompilerParams(use_tc_tiling_on_sc=False),
    )
    def sc_apply(p_hbm, src3_hbm, dst3_hbm, out_hbm, src_idx, dst_idx, rows,
                 zbuf, acc, g0, g1, s0, s1, d0, d1, e0, e1):
        c = lax.axis_index("c")
        sub = lax.axis_index("s")
        base = sub * (K0 + K1) + c * K0
        n = jnp.where(c == 0, K0, K1)
        gsem = (g0, g1)
        ssem = (s0, s1)
        dsem = (d0, d1)
        esem = (e0, e1)

        def estart(ci, b):
            pltpu.async_copy(src3_hbm.at[base + ci], src_idx.at[b], esem[b])

        def ewait(b):
            pltpu.make_async_copy(
                src3_hbm.at[0], src_idx.at[b], esem[b]
            ).wait()

        def gstart(b):
            pltpu.async_copy(p_hbm.at[src_idx.at[b, 0]], rows.at[b], gsem[b])

        def gwait(b):
            pltpu.make_async_copy(
                p_hbm.at[src_idx.at[0, 0]], rows.at[b], gsem[b]
            ).wait()

        def dstart(ci, b):
            pltpu.async_copy(dst3_hbm.at[base + ci], dst_idx.at[b], dsem[b])

        def dwait(b):
            pltpu.make_async_copy(
                dst3_hbm.at[0], dst_idx.at[b], dsem[b]
            ).wait()

        def sstart(b):
            pltpu.async_copy(rows.at[b], acc.at[dst_idx.at[b, 0]], ssem[b],
                             add=True)

        def swait(b):
            pltpu.make_async_copy(
                rows.at[b], acc.at[dst_idx.at[b, 0]], ssem[b]
            ).wait()

        nz = (ZROWS * D) // 16

        def zfill(i, _):
            r = i // (D // 16)
            col = (i % (D // 16)) * 16
            zbuf[r, pl.ds(col, 16)] = jnp.zeros((16,), jnp.float32)
            return 0

        lax.fori_loop(0, nz, zfill, 0)

        estart(0, 0)
        estart(1, 1)
        dstart(0, 0)
        dstart(1, 1)
        ewait(0)
        gstart(0)
        ewait(1)
        gstart(1)

        def zcopy(i, _):
            pltpu.sync_copy(
                zbuf, acc.at[pl.ds(sub * ROWS_PER_TILE + i * ZROWS, ZROWS)]
            )
            return 0

        lax.fori_loop(0, ROWS_PER_TILE // ZROWS, zcopy, 0)
        plsc.subcore_barrier()

        def stage(i, b, more):
            gwait(b)
            dwait(b)
            sstart(b)
            swait(b)
            if more is not False:
                estart(i + 2, b)
                dstart(i + 2, b)
                ewait(b)
                gstart(b)

        def step(k, _):
            i = 2 * k
            stage(i, 0, True)
            stage(i + 1, 1, True)
            return 0

        lax.fori_loop(0, n // 2 - 2, step, 0)
        t = n - 4
        stage(t, 0, True)
        stage(t + 1, 1, True)
        stage(t + 2, 0, False)
        stage(t + 3, 1, False)
        plsc.subcore_barrier()
        pltpu.sync_copy(
            acc.at[pl.ds(sub * ROWS_PER_TILE, ROWS_PER_TILE)],
            out_hbm.at[c, pl.ds(sub * ROWS_PER_TILE, ROWS_PER_TILE)],
        )

    return sc_apply


_sc_apply = {D: _make_sc_apply(D) for D in (128, 64, 32)}


_RB = 1280
_GRID = NPAD // _RB


def _tc0_body(x_ref, w_ref, d0_ref, d1_ref, p_ref, dis_ref):
    dis = lax.rsqrt(d0_ref[...] + d1_ref[...] + 1.0)
    h = jnp.dot(x_ref[...], w_ref[...], preferred_element_type=jnp.float32)
    p_ref[...] = dis * h
    dis_ref[...] = dis


def _tc0(xpad, W1, deg0, deg1):
    D = W1.shape[1]
    return pl.pallas_call(
        _tc0_body,
        grid=(_GRID,),
        in_specs=[
            pl.BlockSpec((_RB, xpad.shape[1]), lambda i: (i, 0)),
            pl.BlockSpec(W1.shape, lambda i: (0, 0)),
            pl.BlockSpec((_RB, 1), lambda i: (i, 0)),
            pl.BlockSpec((_RB, 1), lambda i: (i, 0)),
        ],
        out_specs=[
            pl.BlockSpec((_RB, D), lambda i: (i, 0)),
            pl.BlockSpec((_RB, 1), lambda i: (i, 0)),
        ],
        out_shape=[
            jax.ShapeDtypeStruct((NPAD, D), jnp.float32),
            jax.ShapeDtypeStruct((NPAD, 1), jnp.float32),
        ],
    )(xpad, W1, deg0, deg1)


def _tc_layer_body(s0_ref, s1_ref, p_ref, dis_ref, b_ref, w_ref, out_ref):
    dis = dis_ref[...]
    a = dis * (s0_ref[...] + s1_ref[...] + p_ref[...]) + b_ref[...]
    h = jnp.maximum(a, 0.0)
    out_ref[...] = dis * jnp.dot(
        h, w_ref[...], preferred_element_type=jnp.float32
    )


def _tc_layer(s0, s1, p, dis, b, Wn):
    Din, Dout = Wn.shape
    return pl.pallas_call(
        _tc_layer_body,
        grid=(_GRID,),
        in_specs=[
            pl.BlockSpec((_RB, Din), lambda i: (i, 0)),
            pl.BlockSpec((_RB, Din), lambda i: (i, 0)),
            pl.BlockSpec((_RB, Din), lambda i: (i, 0)),
            pl.BlockSpec((_RB, 1), lambda i: (i, 0)),
            pl.BlockSpec((1, Din), lambda i: (0, 0)),
            pl.BlockSpec((Din, Dout), lambda i: (0, 0)),
        ],
        out_specs=pl.BlockSpec((_RB, Dout), lambda i: (i, 0)),
        out_shape=jax.ShapeDtypeStruct((NPAD, Dout), jnp.float32),
    )(s0, s1, p, dis, b.reshape(1, Din), Wn)


def _tc_final_body(s0_ref, s1_ref, p_ref, dis_ref, b_ref, out_ref):
    out_ref[...] = (
        dis_ref[...] * (s0_ref[...] + s1_ref[...] + p_ref[...]) + b_ref[...]
    )


def _tc_final(s0, s1, p, dis, b):
    D = p.shape[1]
    return pl.pallas_call(
        _tc_final_body,
        grid=(_GRID,),
        in_specs=[
            pl.BlockSpec((_RB, D), lambda i: (i, 0)),
            pl.BlockSpec((_RB, D), lambda i: (i, 0)),
            pl.BlockSpec((_RB, D), lambda i: (i, 0)),
            pl.BlockSpec((_RB, 1), lambda i: (i, 0)),
            pl.BlockSpec((1, D), lambda i: (0, 0)),
        ],
        out_specs=pl.BlockSpec((_RB, D), lambda i: (i, 0)),
        out_shape=jax.ShapeDtypeStruct((NPAD, D), jnp.float32),
    )(s0, s1, p, dis, b.reshape(1, D))


def kernel(x, edge_index, W1, b1, W2, b2, W3, b3, W4, b4):
    extra = EPAD - E
    src3 = jnp.concatenate(
        [edge_index[0], jnp.zeros((extra,), jnp.int32)]
    ).reshape(EPAD // CHUNK, 1, CHUNK)
    trash = N + (jnp.arange(extra, dtype=jnp.int32) % (NPAD - N))
    dst_flat = jnp.concatenate([edge_index[1], trash])
    dst = dst_flat.reshape(EPAD // CHUNK, CHUNK)
    dst3 = dst_flat.reshape(EPAD // CHUNK, 1, CHUNK)
    xpad = jnp.pad(x, ((0, NPAD - N), (0, 0)))

    degp = _deg_kernel(dst)
    deg0 = degp[0].reshape(NPAD, 1)
    deg1 = degp[1].reshape(NPAD, 1)

    p1, dis = _tc0(xpad, W1, deg0, deg1)
    s1 = _sc_apply[128](p1, src3, dst3)
    p2 = _tc_layer(s1[0], s1[1], p1, dis, b1, W2)
    s2 = _sc_apply[128](p2, src3, dst3)
    p3 = _tc_layer(s2[0], s2[1], p2, dis, b2, W3)
    s3 = _sc_apply[64](p3, src3, dst3)
    p4 = _tc_layer(s3[0], s3[1], p3, dis, b3, W4)
    s4 = _sc_apply[32](p4, src3, dst3)
    z = _tc_final(s4[0], s4[1], p4, dis, b4)
    return z[:N]


# speedup vs baseline: 1.3000x; 1.0183x over previous
import functools

import jax
import jax.numpy as jnp
from jax import lax
from jax.experimental import pallas as pl
from jax.experimental.pallas import tpu as pltpu
from jax.experimental.pallas import tpu_sc as plsc

N = 10000
E = 320000
NPAD = 10240

NC = 2
NS = 16
NW = NC * NS
CHUNK = 128
NCHUNK = 80
EP = NCHUNK * CHUNK
EPAD = NW * EP
ZROWS = 16
ROWS_PER_TILE = NPAD // NS
K0 = 120
K1 = 40

_MESH = plsc.VectorSubcoreMesh(core_axis_name="c", subcore_axis_name="s")


@functools.partial(
    pl.kernel,
    out_type=jax.ShapeDtypeStruct((NC, NPAD), jnp.float32),
    mesh=_MESH,
    scratch_types=[
        pltpu.VMEM((NCHUNK, CHUNK), jnp.int32),
        pltpu.VMEM((CHUNK,), jnp.float32),
        pltpu.VMEM((ROWS_PER_TILE,), jnp.float32),
        pltpu.VMEM_SHARED((NPAD,), jnp.float32),
        pltpu.SemaphoreType.DMA,
    ],
)
def _deg_kernel(dst_hbm, out_hbm, dst_idx, ones_v, zbuf, acc, isem):
    c = lax.axis_index("c")
    sub = lax.axis_index("s")
    wid = sub * NC + c

    icp = pltpu.async_copy(dst_hbm.at[pl.ds(wid * NCHUNK, NCHUNK)], dst_idx,
                           isem)

    def fill(i, _):
        ones_v[pl.ds(i * 16, 16)] = jnp.ones((16,), jnp.float32)
        return 0

    lax.fori_loop(0, CHUNK // 16, fill, 0)

    def zfill(i, _):
        zbuf[pl.ds(i * 16, 16)] = jnp.zeros((16,), jnp.float32)
        return 0

    lax.fori_loop(0, ROWS_PER_TILE // 16, zfill, 0)
    pltpu.sync_copy(zbuf, acc.at[pl.ds(sub * ROWS_PER_TILE, ROWS_PER_TILE)])
    icp.wait()
    plsc.subcore_barrier()

    def body(i, _):
        pltpu.async_copy(ones_v, acc.at[dst_idx.at[i]], isem, add=True)
        return 0

    lax.fori_loop(0, NCHUNK, body, 0)

    def drain(i, _):
        pltpu.make_async_copy(ones_v, acc.at[dst_idx.at[0]], isem).wait()
        return 0

    lax.fori_loop(0, NCHUNK, drain, 0)
    plsc.subcore_barrier()
    pltpu.sync_copy(
        acc.at[pl.ds(sub * ROWS_PER_TILE, ROWS_PER_TILE)],
        out_hbm.at[c, pl.ds(sub * ROWS_PER_TILE, ROWS_PER_TILE)],
    )


def _make_sc_apply(D):
    @functools.partial(
        pl.kernel,
        out_type=jax.ShapeDtypeStruct((NC, NPAD, D), jnp.float32),
        mesh=_MESH,
        scratch_types=[
            pltpu.VMEM((4, 1, CHUNK), jnp.int32),
            pltpu.VMEM((4, 1, CHUNK), jnp.int32),
            pltpu.VMEM((2, CHUNK, D), jnp.float32),
            pltpu.VMEM((ZROWS, D), jnp.float32),
            pltpu.VMEM_SHARED((NPAD, D), jnp.float32),
            [pltpu.SemaphoreType.DMA] * 12,
        ],
        compiler_params=pltpu.CompilerParams(use_tc_tiling_on_sc=False),
    )
    def sc_apply(p_hbm, src3_hbm, dst3_hbm, out_hbm, src_idx, dst_idx, rows,
                 zbuf, acc, sems):
        c = lax.axis_index("c")
        sub = lax.axis_index("s")
        base = sub * (K0 + K1) + c * K0
        n = jnp.where(c == 0, K0, K1)
        gsem = sems[0:2]
        ssem = sems[2:4]
        esem = sems[4:8]
        dsem = sems[8:12]

        def estart(ci, ib):
            pltpu.async_copy(src3_hbm.at[base + ci], src_idx.at[ib],
                             esem[ib])

        def ewait(ib):
            pltpu.make_async_copy(
                src3_hbm.at[0], src_idx.at[ib], esem[ib]
            ).wait()

        def gstart(b, ib):
            pltpu.async_copy(p_hbm.at[src_idx.at[ib, 0]], rows.at[b],
                             gsem[b])

        def gwait(b):
            pltpu.make_async_copy(
                p_hbm.at[src_idx.at[0, 0]], rows.at[b], gsem[b]
            ).wait()

        def dstart(ci, ib):
            pltpu.async_copy(dst3_hbm.at[base + ci], dst_idx.at[ib],
                             dsem[ib])

        def dwait(ib):
            pltpu.make_async_copy(
                dst3_hbm.at[0], dst_idx.at[ib], dsem[ib]
            ).wait()

        def sstart(b, ib):
            pltpu.async_copy(rows.at[b], acc.at[dst_idx.at[ib, 0]], ssem[b],
                             add=True)

        def swait(b):
            pltpu.make_async_copy(
                rows.at[b], acc.at[dst_idx.at[0, 0]], ssem[b]
            ).wait()

        nz = (ZROWS * D) // 16

        def zfill(i, _):
            r = i // (D // 16)
            col = (i % (D // 16)) * 16
            zbuf[r, pl.ds(col, 16)] = jnp.zeros((16,), jnp.float32)
            return 0

        lax.fori_loop(0, nz, zfill, 0)

        for j in range(4):
            estart(j, j)
            dstart(j, j)
        ewait(0)
        gstart(0, 0)
        ewait(1)
        gstart(1, 1)

        def zcopy(i, _):
            pltpu.sync_copy(
                zbuf, acc.at[pl.ds(sub * ROWS_PER_TILE + i * ZROWS, ZROWS)]
            )
            return 0

        lax.fori_loop(0, ROWS_PER_TILE // ZROWS, zcopy, 0)
        plsc.subcore_barrier()

        def stage(i, j, refill, prefetch):
            b = j % 2
            ib = j % 4
            ib2 = (j + 2) % 4
            gwait(b)
            dwait(ib)
            sstart(b, ib)
            swait(b)
            if refill:
                estart(i + 4, ib)
                dstart(i + 4, ib)
            if prefetch:
                ewait(ib2)
                gstart(b, ib2)

        def step(k, _):
            i = 4 * k
            for j in range(4):
                stage(i + j, j, True, True)
            return 0

        lax.fori_loop(0, n // 4 - 2, step, 0)
        t = n - 8
        for j in range(4):
            stage(t + j, j, True, True)
        stage(t + 4, 0, False, True)
        stage(t + 5, 1, False, True)
        stage(t + 6, 2, False, False)
        stage(t + 7, 3, False, False)
        plsc.subcore_barrier()
        pltpu.sync_copy(
            acc.at[pl.ds(sub * ROWS_PER_TILE, ROWS_PER_TILE)],
            out_hbm.at[c, pl.ds(sub * ROWS_PER_TILE, ROWS_PER_TILE)],
        )

    return sc_apply


_sc_apply = {D: _make_sc_apply(D) for D in (128, 64, 32)}


_RB = 1280
_GRID = NPAD // _RB


def _tc0_body(x_ref, w_ref, d0_ref, d1_ref, p_ref, dis_ref):
    dis = lax.rsqrt(d0_ref[...] + d1_ref[...] + 1.0)
    h = jnp.dot(x_ref[...], w_ref[...], preferred_element_type=jnp.float32)
    p_ref[...] = dis * h
    dis_ref[...] = dis


def _tc0(xpad, W1, deg0, deg1):
    D = W1.shape[1]
    return pl.pallas_call(
        _tc0_body,
        grid=(_GRID,),
        in_specs=[
            pl.BlockSpec((_RB, xpad.shape[1]), lambda i: (i, 0)),
            pl.BlockSpec(W1.shape, lambda i: (0, 0)),
            pl.BlockSpec((_RB, 1), lambda i: (i, 0)),
            pl.BlockSpec((_RB, 1), lambda i: (i, 0)),
        ],
        out_specs=[
            pl.BlockSpec((_RB, D), lambda i: (i, 0)),
            pl.BlockSpec((_RB, 1), lambda i: (i, 0)),
        ],
        out_shape=[
            jax.ShapeDtypeStruct((NPAD, D), jnp.float32),
            jax.ShapeDtypeStruct((NPAD, 1), jnp.float32),
        ],
    )(xpad, W1, deg0, deg1)


def _tc_layer_body(s0_ref, s1_ref, p_ref, dis_ref, b_ref, w_ref, out_ref):
    dis = dis_ref[...]
    a = dis * (s0_ref[...] + s1_ref[...] + p_ref[...]) + b_ref[...]
    h = jnp.maximum(a, 0.0)
    out_ref[...] = dis * jnp.dot(
        h, w_ref[...], preferred_element_type=jnp.float32
    )


def _tc_layer(s0, s1, p, dis, b, Wn):
    Din, Dout = Wn.shape
    return pl.pallas_call(
        _tc_layer_body,
        grid=(_GRID,),
        in_specs=[
            pl.BlockSpec((_RB, Din), lambda i: (i, 0)),
            pl.BlockSpec((_RB, Din), lambda i: (i, 0)),
            pl.BlockSpec((_RB, Din), lambda i: (i, 0)),
            pl.BlockSpec((_RB, 1), lambda i: (i, 0)),
            pl.BlockSpec((1, Din), lambda i: (0, 0)),
            pl.BlockSpec((Din, Dout), lambda i: (0, 0)),
        ],
        out_specs=pl.BlockSpec((_RB, Dout), lambda i: (i, 0)),
        out_shape=jax.ShapeDtypeStruct((NPAD, Dout), jnp.float32),
    )(s0, s1, p, dis, b.reshape(1, Din), Wn)


def _tc_final_body(s0_ref, s1_ref, p_ref, dis_ref, b_ref, out_ref):
    out_ref[...] = (
        dis_ref[...] * (s0_ref[...] + s1_ref[...] + p_ref[...]) + b_ref[...]
    )


def _tc_final(s0, s1, p, dis, b):
    D = p.shape[1]
    return pl.pallas_call(
        _tc_final_body,
        grid=(_GRID,),
        in_specs=[
            pl.BlockSpec((_RB, D), lambda i: (i, 0)),
            pl.BlockSpec((_RB, D), lambda i: (i, 0)),
            pl.BlockSpec((_RB, D), lambda i: (i, 0)),
            pl.BlockSpec((_RB, 1), lambda i: (i, 0)),
            pl.BlockSpec((1, D), lambda i: (0, 0)),
        ],
        out_specs=pl.BlockSpec((_RB, D), lambda i: (i, 0)),
        out_shape=jax.ShapeDtypeStruct((NPAD, D), jnp.float32),
    )(s0, s1, p, dis, b.reshape(1, D))


def kernel(x, edge_index, W1, b1, W2, b2, W3, b3, W4, b4):
    extra = EPAD - E
    src3 = jnp.concatenate(
        [edge_index[0], jnp.zeros((extra,), jnp.int32)]
    ).reshape(EPAD // CHUNK, 1, CHUNK)
    trash = N + (jnp.arange(extra, dtype=jnp.int32) % (NPAD - N))
    dst_flat = jnp.concatenate([edge_index[1], trash])
    dst = dst_flat.reshape(EPAD // CHUNK, CHUNK)
    dst3 = dst_flat.reshape(EPAD // CHUNK, 1, CHUNK)
    xpad = jnp.pad(x, ((0, NPAD - N), (0, 0)))

    degp = _deg_kernel(dst)
    deg0 = degp[0].reshape(NPAD, 1)
    deg1 = degp[1].reshape(NPAD, 1)

    p1, dis = _tc0(xpad, W1, deg0, deg1)
    s1 = _sc_apply[128](p1, src3, dst3)
    p2 = _tc_layer(s1[0], s1[1], p1, dis, b1, W2)
    s2 = _sc_apply[128](p2, src3, dst3)
    p3 = _tc_layer(s2[0], s2[1], p2, dis, b2, W3)
    s3 = _sc_apply[64](p3, src3, dst3)
    p4 = _tc_layer(s3[0], s3[1], p3, dis, b3, W4)
    s4 = _sc_apply[32](p4, src3, dst3)
    z = _tc_final(s4[0], s4[1], p4, dis, b4)
    return z[:N]
